# baseline (speedup 1.0000x reference)
import functools

import jax
import jax.numpy as jnp
from jax import lax
from jax.experimental import pallas as pl
from jax.experimental.pallas import tpu as pltpu
from jax.experimental.pallas import tpu_sc as plsc

_NUM_CODES = 1024
_COMMIT = 0.25
_BM = 1024


def _dist_argmin_body(x_ref, w_ref, idx_ref, loss_ref):
    xb = x_ref[...]
    w = w_ref[...]
    scores = lax.dot_general(w, xb, (((1,), (1,)), ((), ())),
                             preferred_element_type=jnp.float32)
    bm = xb.shape[0]
    x_sq = jnp.sum(xb * xb, axis=1)[None, :]
    w_sq = jnp.sum(w * w, axis=1)
    best_v = best_i = None
    for j in range(_NUM_CODES // 8):
        sj = scores[8 * j:8 * j + 8, :]
        wsj = w_sq[8 * j:8 * j + 8][:, None]
        d2 = (x_sq - 2.0 * sj) + wsj
        dist = jnp.sqrt(jnp.clip(d2, 0.0, None))
        if j == 0:
            best_v = dist
            best_i = jnp.zeros((8, bm), jnp.int32)
        else:
            lt = dist < best_v
            best_v = jnp.minimum(best_v, dist)
            best_i = jnp.where(lt, jnp.int32(j), best_i)
    sub = lax.broadcasted_iota(jnp.int32, (8, bm), 0)
    full_i = best_i * 8 + sub
    m = jnp.min(best_v, axis=0, keepdims=True)
    idx = jnp.min(jnp.where(best_v == m, full_i, jnp.int32(_NUM_CODES)),
                  axis=0)
    idx_ref[...] = idx
    part = jnp.sum(m * m, axis=1, keepdims=True)

    @pl.when(pl.program_id(0) == 0)
    def _():
        loss_ref[...] = jnp.zeros((1, 1), jnp.float32)

    loss_ref[...] += part


def _encode(flat_x, W):
    n_rows = flat_x.shape[0]
    grid = n_rows // _BM
    return pl.pallas_call(
        _dist_argmin_body,
        grid=(grid,),
        in_specs=[
            pl.BlockSpec((_BM, flat_x.shape[1]), lambda i: (i, 0)),
            pl.BlockSpec(W.shape, lambda i: (0, 0)),
        ],
        out_specs=[
            pl.BlockSpec((_BM,), lambda i: (i,)),
            pl.BlockSpec((1, 1), lambda i: (0, 0)),
        ],
        out_shape=[
            jax.ShapeDtypeStruct((n_rows,), jnp.int32),
            jax.ShapeDtypeStruct((1, 1), jnp.float32),
        ],
    )(flat_x, W)


def _sc_gather(table, idx):
    B = idx.shape[0]
    D = table.shape[1]
    NC, NS = 2, 16
    b_per_w = B // (NC * NS)
    mesh = plsc.VectorSubcoreMesh(core_axis_name="c", subcore_axis_name="s")

    @functools.partial(
        pl.kernel, mesh=mesh,
        compiler_params=pltpu.CompilerParams(use_tc_tiling_on_sc=False),
        out_type=jax.ShapeDtypeStruct((B, D), jnp.float32),
        scratch_types=[
            pltpu.VMEM((b_per_w,), jnp.int32),
            pltpu.VMEM((b_per_w, D), jnp.float32),
            pltpu.SemaphoreType.DMA,
        ],
    )
    def gather_k(table_hbm, idx_hbm, out_hbm, idx_v, rows_v, sem):
        wid = lax.axis_index("s") * NC + lax.axis_index("c")
        base = wid * b_per_w
        pltpu.sync_copy(idx_hbm.at[pl.ds(base, b_per_w)], idx_v)
        pltpu.async_copy(table_hbm.at[idx_v], rows_v, sem).wait()
        pltpu.sync_copy(rows_v, out_hbm.at[pl.ds(base, b_per_w)])

    return gather_k(table, idx)


def kernel(x, W):
    batch, seq, dim = x.shape
    flat_x = x.reshape(batch * seq, dim)
    idx, loss = _encode(flat_x, W)
    q = _sc_gather(W, idx)
    quantizar_st = q.reshape(batch, seq, dim)
    n_el = batch * seq * dim
    perda = loss[0, 0] * ((1.0 + _COMMIT) / n_el)
    return quantizar_st, perda


# speedup vs baseline: 32.6584x; 32.6584x over previous
import functools

import jax
import jax.numpy as jnp
from jax import lax
from jax.experimental import pallas as pl
from jax.experimental.pallas import tpu as pltpu
from jax.experimental.pallas import tpu_sc as plsc

_NUM_CODES = 1024
_COMMIT = 0.25
_BM = 1024


def _dist_argmin_body(x_ref, w_ref, idx_ref, loss_ref):
    xb = x_ref[...]
    w = w_ref[...]
    dot = lax.dot_general(xb, w, (((1,), (1,)), ((), ())),
                          preferred_element_type=jnp.float32)
    x_sq = jnp.sum(xb * xb, axis=1, keepdims=True)
    w_sq = jnp.sum(w * w, axis=1)[None, :]
    d2 = (x_sq - 2.0 * dot) + w_sq
    dist = jnp.sqrt(jnp.clip(d2, 0.0, None))
    minv = jnp.min(dist, axis=1, keepdims=True)
    ids = lax.broadcasted_iota(jnp.int32, (1, _NUM_CODES), 1).astype(jnp.float32)
    idx_f = jnp.min(jnp.where(dist == minv, ids, jnp.float32(_NUM_CODES)),
                    axis=1)
    idx_ref[...] = idx_f.astype(jnp.int32)
    part = jnp.sum(minv * minv, keepdims=True)

    @pl.when(pl.program_id(0) == 0)
    def _():
        loss_ref[...] = jnp.zeros((1, 1), jnp.float32)

    loss_ref[...] += part


def _encode(flat_x, W):
    n_rows, dim = flat_x.shape
    grid = n_rows // _BM
    return pl.pallas_call(
        _dist_argmin_body,
        grid=(grid,),
        in_specs=[
            pl.BlockSpec((_BM, dim), lambda i: (i, 0)),
            pl.BlockSpec(W.shape, lambda i: (0, 0)),
        ],
        out_specs=[
            pl.BlockSpec((_BM,), lambda i: (i,)),
            pl.BlockSpec((1, 1), lambda i: (0, 0)),
        ],
        out_shape=[
            jax.ShapeDtypeStruct((n_rows,), jnp.int32),
            jax.ShapeDtypeStruct((1, 1), jnp.float32),
        ],
    )(flat_x, W)


def _sc_gather(table, idx):
    B = idx.shape[0]
    D = table.shape[1]
    NC, NS = 2, 16
    b_per_w = B // (NC * NS)
    mesh = plsc.VectorSubcoreMesh(core_axis_name="c", subcore_axis_name="s")

    @functools.partial(
        pl.kernel, mesh=mesh,
        compiler_params=pltpu.CompilerParams(use_tc_tiling_on_sc=False),
        out_type=jax.ShapeDtypeStruct((B, D), jnp.float32),
        scratch_types=[
            pltpu.VMEM((b_per_w,), jnp.int32),
            pltpu.VMEM((b_per_w, D), jnp.float32),
            pltpu.SemaphoreType.DMA,
        ],
    )
    def gather_k(table_hbm, idx_hbm, out_hbm, idx_v, rows_v, sem):
        wid = lax.axis_index("s") * NC + lax.axis_index("c")
        base = wid * b_per_w
        pltpu.sync_copy(idx_hbm.at[pl.ds(base, b_per_w)], idx_v)
        pltpu.async_copy(table_hbm.at[idx_v], rows_v, sem).wait()
        pltpu.sync_copy(rows_v, out_hbm.at[pl.ds(base, b_per_w)])

    return gather_k(table, idx)


def kernel(x, W):
    batch, seq, dim = x.shape
    flat_x = x.reshape(batch * seq, dim)
    idx, loss = _encode(flat_x, W)
    q = _sc_gather(W, idx)
    quantizar_st = q.reshape(batch, seq, dim)
    n_el = batch * seq * dim
    perda = loss[0, 0] * ((1.0 + _COMMIT) / n_el)
    return quantizar_st, perda
